# Initial kernel scaffold; baseline (speedup 1.0000x reference)
import jax
import jax.numpy as jnp
from jax.experimental import pallas as pl


def kernel(logprobs, beam_logprobs_sum, beam_seq):
    raise NotImplementedError("write your pallas kernel here")


# speedup vs baseline: 31.0223x; 31.0223x over previous
import jax
import jax.numpy as jnp
from jax import lax
from jax.experimental import pallas as pl
from jax.experimental.pallas import tpu as pltpu

_BATCH = 64
_BEAM = 10
_VOCAB = 100000
_TPREV = 5
_LANES = 16


def _beam_step_body(lp_ref, bls_ref, seqt_ref, seq_out_ref, bls_out_ref):
    x = lp_ref[0]
    bls = bls_ref[0]
    seqt = seqt_ref[0]

    m = jnp.max(x, axis=1, keepdims=True)
    s = jnp.sum(jnp.exp(x - m), axis=1, keepdims=True)
    shift = bls - jnp.log(s)
    c = (x - m) + shift

    flat = (
        lax.broadcasted_iota(jnp.int32, (_BEAM, _VOCAB), 0) * _VOCAB
        + lax.broadcasted_iota(jnp.int32, (_BEAM, _VOCAB), 1)
    )

    lane = lax.broadcasted_iota(jnp.int32, (1, _LANES), 1)
    vals = jnp.zeros((1, _LANES), jnp.float32)
    fidx = jnp.zeros((1, _LANES), jnp.int32)
    big = jnp.int32(2**31 - 1)
    for i in range(_BEAM):
        gm = jnp.max(c, keepdims=True)
        fi = jnp.min(jnp.where(c == gm, flat, big), keepdims=True)
        vals = jnp.where(lane == i, gm, vals)
        fidx = jnp.where(lane == i, fi, fidx)
        if i + 1 < _BEAM:
            c = jnp.where(flat == fi, -jnp.inf, c)

    beam_ix = fidx // _VOCAB
    sel_ix = fidx % _VOCAB

    new_prev_i = jnp.zeros((_TPREV, _LANES), jnp.int32)
    for k in range(_BEAM):
        col = seqt[:, k : k + 1]
        new_prev_i = jnp.where(beam_ix == k, col, new_prev_i)

    seq_out_ref[0] = jnp.concatenate([new_prev_i, sel_ix], axis=0)
    bls_out_ref[0] = vals


def kernel(logprobs, beam_logprobs_sum, beam_seq):
    lp = logprobs.reshape(_BATCH, _BEAM, _VOCAB)
    bls = beam_logprobs_sum.reshape(_BATCH, _BEAM, 1)
    seqt = beam_seq.transpose(0, 2, 1)

    seq_out, bls_out = pl.pallas_call(
        _beam_step_body,
        grid=(_BATCH,),
        in_specs=[
            pl.BlockSpec((1, _BEAM, _VOCAB), lambda b: (b, 0, 0)),
            pl.BlockSpec((1, _BEAM, 1), lambda b: (b, 0, 0)),
            pl.BlockSpec((1, _TPREV, _BEAM), lambda b: (b, 0, 0)),
        ],
        out_specs=[
            pl.BlockSpec((1, _TPREV + 1, _LANES), lambda b: (b, 0, 0)),
            pl.BlockSpec((1, 1, _LANES), lambda b: (b, 0, 0)),
        ],
        out_shape=[
            jax.ShapeDtypeStruct((_BATCH, _TPREV + 1, _LANES), jnp.int32),
            jax.ShapeDtypeStruct((_BATCH, 1, _LANES), jnp.float32),
        ],
    )(lp, bls, seqt)

    new_seq = seq_out.transpose(0, 2, 1)[:, :_BEAM, :]
    new_bls = bls_out[:, 0, :_BEAM]
    return new_seq, new_bls


# speedup vs baseline: 48.3544x; 1.5587x over previous
import jax
import jax.numpy as jnp
from jax import lax
from jax.experimental import pallas as pl
from jax.experimental.pallas import tpu as pltpu

_BATCH = 64
_BEAM = 10
_VOCAB = 100000
_TPREV = 5
_LANES = 16
_RS = 784
_VPAD = _RS * 128


def _beam_step_body(lp_ref, bls_ref, seqt_ref, seq_out_ref, bls_out_ref):
    x = lp_ref[0]
    bls = bls_ref[0]
    seqt = seqt_ref[0]

    big = jnp.int32(2**31 - 1)
    neg = jnp.float32(-jnp.inf)

    p0 = jnp.max(x, axis=1)
    m = jnp.max(p0, axis=1, keepdims=True)
    e = jnp.exp(x - m[:, :, None])
    s = jnp.sum(jnp.sum(e, axis=1), axis=1, keepdims=True)
    shift = bls - jnp.log(s)

    cp = (p0 - m) + shift

    vflat = (
        lax.broadcasted_iota(jnp.int32, (_BEAM, _RS, 128), 1) * 128
        + lax.broadcasted_iota(jnp.int32, (_BEAM, _RS, 128), 2)
    )
    cf = jnp.min(
        jnp.where(x == p0[:, None, :], vflat, big), axis=1
    )
    cff = lax.broadcasted_iota(jnp.int32, (_BEAM, 128), 0) * _VOCAB + cf

    lane16 = lax.broadcasted_iota(jnp.int32, (1, _LANES), 1)
    beam_pc = lax.broadcasted_iota(jnp.int32, (_BEAM, 128), 0)
    lane_pc = lax.broadcasted_iota(jnp.int32, (_BEAM, 128), 1)
    beam10 = lax.broadcasted_iota(jnp.int32, (_BEAM, 1), 0)
    lane_slab = lax.broadcasted_iota(jnp.int32, (_RS, 128), 1)
    vflat_slab = (
        lax.broadcasted_iota(jnp.int32, (_RS, 128), 0) * 128
        + lax.broadcasted_iota(jnp.int32, (_RS, 128), 1)
    )

    vals = jnp.zeros((1, _LANES), jnp.float32)
    fidx = jnp.zeros((1, _LANES), jnp.int32)
    for i in range(_BEAM):
        gm = jnp.max(cp)
        fi = jnp.min(jnp.where(cp == gm, cff, big))
        vals = jnp.where(lane16 == i, gm, vals)
        fidx = jnp.where(lane16 == i, fi, fidx)
        if i + 1 == _BEAM:
            break

        kstar = fi // _VOCAB
        lstar = (fi % _VOCAB) % 128
        ksel = beam10 == kstar
        m_k = jnp.sum(jnp.where(ksel, m, 0.0))
        sh_k = jnp.sum(jnp.where(ksel, shift, 0.0))

        slab = lp_ref[0, kstar]
        cs = (slab - m_k) + sh_k
        ff = kstar * _VOCAB + vflat_slab
        elig = (lane_slab == lstar) & ((cs < gm) | ((cs == gm) & (ff > fi)))
        newv = jnp.max(jnp.where(elig, cs, neg))
        newf = jnp.min(jnp.where(elig & (cs == newv), ff, big))

        colmask = (beam_pc == kstar) & (lane_pc == lstar)
        cp = jnp.where(colmask, newv, cp)
        cff = jnp.where(colmask, newf, cff)

    beam_ix = fidx // _VOCAB
    sel_ix = fidx % _VOCAB

    new_prev_i = jnp.zeros((_TPREV, _LANES), jnp.int32)
    for k in range(_BEAM):
        col = seqt[:, k : k + 1]
        new_prev_i = jnp.where(beam_ix == k, col, new_prev_i)

    seq_out_ref[0] = jnp.concatenate([new_prev_i, sel_ix], axis=0)
    bls_out_ref[0] = vals


def kernel(logprobs, beam_logprobs_sum, beam_seq):
    lp = logprobs.reshape(_BATCH, _BEAM, _VOCAB)
    lp4 = jnp.pad(
        lp, ((0, 0), (0, 0), (0, _VPAD - _VOCAB)), constant_values=-jnp.inf
    ).reshape(_BATCH, _BEAM, _RS, 128)
    bls = beam_logprobs_sum.reshape(_BATCH, _BEAM, 1)
    seqt = beam_seq.transpose(0, 2, 1)

    seq_out, bls_out = pl.pallas_call(
        _beam_step_body,
        grid=(_BATCH,),
        in_specs=[
            pl.BlockSpec((1, _BEAM, _RS, 128), lambda b: (b, 0, 0, 0)),
            pl.BlockSpec((1, _BEAM, 1), lambda b: (b, 0, 0)),
            pl.BlockSpec((1, _TPREV, _BEAM), lambda b: (b, 0, 0)),
        ],
        out_specs=[
            pl.BlockSpec((1, _TPREV + 1, _LANES), lambda b: (b, 0, 0)),
            pl.BlockSpec((1, 1, _LANES), lambda b: (b, 0, 0)),
        ],
        out_shape=[
            jax.ShapeDtypeStruct((_BATCH, _TPREV + 1, _LANES), jnp.int32),
            jax.ShapeDtypeStruct((_BATCH, 1, _LANES), jnp.float32),
        ],
    )(lp4, bls, seqt)

    new_seq = seq_out.transpose(0, 2, 1)[:, :_BEAM, :]
    new_bls = bls_out[:, 0, :_BEAM]
    return new_seq, new_bls


# speedup vs baseline: 61.8216x; 1.2785x over previous
import jax
import jax.numpy as jnp
from jax import lax
from jax.experimental import pallas as pl
from jax.experimental.pallas import tpu as pltpu

_BATCH = 64
_BEAM = 10
_VOCAB = 100000
_TPREV = 5
_LANES = 16
_RS = 784
_SLOT = _RS * 128
_CPY = 100096


def _beam_step_body(lp_hbm, bls_ref, seqt_ref, seq_out_ref, bls_out_ref, xb, sem):
    b = pl.program_id(0)
    slot = lax.rem(b, 2)
    nslot = lax.rem(b + 1, 2)
    soff = pl.multiple_of(slot * (_BEAM * _SLOT), 1024)

    def copies(batch, sl):
        base = pl.multiple_of(sl * (_BEAM * _SLOT), 1024)
        out = []
        for k in range(_BEAM):
            q = (batch * _BEAM + k) * _VOCAB
            src_off = pl.multiple_of(q - lax.rem(q, 128), 128)
            out.append(
                pltpu.make_async_copy(
                    lp_hbm.at[pl.ds(src_off, _CPY)],
                    xb.at[pl.ds(pl.multiple_of(base + k * _SLOT, 1024), _CPY)],
                    sem.at[sl],
                )
            )
        return out

    @pl.when(b == 0)
    def _():
        for c in copies(b, slot):
            c.start()

    @pl.when(b + 1 < _BATCH)
    def _():
        for c in copies(b + 1, nslot):
            c.start()

    for c in copies(b, slot):
        c.wait()

    x = xb[pl.ds(soff, _BEAM * _SLOT)].reshape(_BEAM, _RS, 128)
    bls = bls_ref[0]
    seqt = seqt_ref[0]

    big = jnp.int32(2**31 - 1)
    neg = jnp.float32(-jnp.inf)

    beam3 = lax.broadcasted_iota(jnp.int32, (_BEAM, 1, 1), 0)
    p_arr = 32 * lax.rem(2 * b + beam3, 4)
    vflat = (
        lax.broadcasted_iota(jnp.int32, (_BEAM, _RS, 128), 1) * 128
        + lax.broadcasted_iota(jnp.int32, (_BEAM, _RS, 128), 2)
        - p_arr
    )
    valid = (vflat >= 0) & (vflat < _VOCAB)

    xm = jnp.where(valid, x, neg)
    p0 = jnp.max(xm, axis=1)
    m = jnp.max(p0, axis=1, keepdims=True)
    e = jnp.where(valid, jnp.exp(x - m[:, :, None]), 0.0)
    s = jnp.sum(jnp.sum(e, axis=1), axis=1, keepdims=True)
    shift = bls - jnp.log(s)

    cp = (p0 - m) + shift

    cf = jnp.min(
        jnp.where(valid & (x == p0[:, None, :]), vflat, big), axis=1
    )
    cff = lax.broadcasted_iota(jnp.int32, (_BEAM, 128), 0) * _VOCAB + cf

    lane16 = lax.broadcasted_iota(jnp.int32, (1, _LANES), 1)
    beam_pc = lax.broadcasted_iota(jnp.int32, (_BEAM, 128), 0)
    lane_pc = lax.broadcasted_iota(jnp.int32, (_BEAM, 128), 1)
    beam10 = lax.broadcasted_iota(jnp.int32, (_BEAM, 1), 0)
    lane_slab = lax.broadcasted_iota(jnp.int32, (_RS, 128), 1)
    rl_slab = (
        lax.broadcasted_iota(jnp.int32, (_RS, 128), 0) * 128
        + lax.broadcasted_iota(jnp.int32, (_RS, 128), 1)
    )

    vals = jnp.zeros((1, _LANES), jnp.float32)
    fidx = jnp.zeros((1, _LANES), jnp.int32)
    for i in range(_BEAM):
        gm = jnp.max(cp)
        fi = jnp.min(jnp.where(cp == gm, cff, big))
        vals = jnp.where(lane16 == i, gm, vals)
        fidx = jnp.where(lane16 == i, fi, fidx)
        if i + 1 == _BEAM:
            break

        kstar = fi // _VOCAB
        vstar = lax.rem(fi, _VOCAB)
        p_k = 32 * lax.rem(2 * b + kstar, 4)
        lstar = lax.rem(vstar + p_k, 128)
        ksel = beam10 == kstar
        m_k = jnp.sum(jnp.where(ksel, m, 0.0))
        sh_k = jnp.sum(jnp.where(ksel, shift, 0.0))

        off = pl.multiple_of(soff + kstar * _SLOT, 1024)
        slab = xb[pl.ds(off, _SLOT)].reshape(_RS, 128)
        vf_s = rl_slab - p_k
        valid_s = (vf_s >= 0) & (vf_s < _VOCAB)
        cs = (slab - m_k) + sh_k
        ff = kstar * _VOCAB + vf_s
        elig = (
            valid_s
            & (lane_slab == lstar)
            & ((cs < gm) | ((cs == gm) & (ff > fi)))
        )
        newv = jnp.max(jnp.where(elig, cs, neg))
        newf = jnp.min(jnp.where(elig & (cs == newv), ff, big))

        colmask = (beam_pc == kstar) & (lane_pc == lstar)
        cp = jnp.where(colmask, newv, cp)
        cff = jnp.where(colmask, newf, cff)

    beam_ix = fidx // _VOCAB
    sel_ix = fidx % _VOCAB

    new_prev_i = jnp.zeros((_TPREV, _LANES), jnp.int32)
    for k in range(_BEAM):
        col = seqt[:, k : k + 1]
        new_prev_i = jnp.where(beam_ix == k, col, new_prev_i)

    seq_out_ref[0] = jnp.concatenate([new_prev_i, sel_ix], axis=0)
    bls_out_ref[0] = vals


def kernel(logprobs, beam_logprobs_sum, beam_seq):
    bls = beam_logprobs_sum.reshape(_BATCH, _BEAM, 1)
    seqt = beam_seq.transpose(0, 2, 1)

    seq_out, bls_out = pl.pallas_call(
        _beam_step_body,
        grid=(_BATCH,),
        in_specs=[
            pl.BlockSpec(memory_space=pl.ANY),
            pl.BlockSpec((1, _BEAM, 1), lambda b: (b, 0, 0)),
            pl.BlockSpec((1, _TPREV, _BEAM), lambda b: (b, 0, 0)),
        ],
        out_specs=[
            pl.BlockSpec((1, _TPREV + 1, _LANES), lambda b: (b, 0, 0)),
            pl.BlockSpec((1, 1, _LANES), lambda b: (b, 0, 0)),
        ],
        out_shape=[
            jax.ShapeDtypeStruct((_BATCH, _TPREV + 1, _LANES), jnp.int32),
            jax.ShapeDtypeStruct((_BATCH, 1, _LANES), jnp.float32),
        ],
        scratch_shapes=[
            pltpu.VMEM((2 * _BEAM * _SLOT,), jnp.float32),
            pltpu.SemaphoreType.DMA((2,)),
        ],
    )(logprobs.reshape(-1), bls, seqt)

    new_seq = seq_out.transpose(0, 2, 1)[:, :_BEAM, :]
    new_bls = bls_out[:, 0, :_BEAM]
    return new_seq, new_bls


# speedup vs baseline: 68.6660x; 1.1107x over previous
import jax
import jax.numpy as jnp
from jax import lax
from jax.experimental import pallas as pl
from jax.experimental.pallas import tpu as pltpu

_BATCH = 64
_BEAM = 10
_VOCAB = 100000
_TPREV = 5
_LANES = 16
_RS = 784
_SLOT = _RS * 128
_CPY = 100096
_NBUF = 4


def _beam_step_body(lp_hbm, bls_ref, seqt_ref, seq_out_ref, bls_out_ref, xb, sem):
    b = pl.program_id(0)
    slot = lax.rem(b, _NBUF)
    soff = pl.multiple_of(slot * (_BEAM * _SLOT), 1024)

    def copies(batch, sl):
        base = pl.multiple_of(sl * (_BEAM * _SLOT), 1024)
        out = []
        for k in range(_BEAM):
            q = (batch * _BEAM + k) * _VOCAB
            src_off = pl.multiple_of(q - lax.rem(q, 128), 128)
            out.append(
                pltpu.make_async_copy(
                    lp_hbm.at[pl.ds(src_off, _CPY)],
                    xb.at[pl.ds(pl.multiple_of(base + k * _SLOT, 1024), _CPY)],
                    sem.at[sl],
                )
            )
        return out

    @pl.when(b == 0)
    def _():
        for c in copies(b, slot) + copies(b + 1, lax.rem(b + 1, _NBUF)):
            c.start()

    @pl.when(b + 2 < _BATCH)
    def _():
        for c in copies(b + 2, lax.rem(b + 2, _NBUF)):
            c.start()

    for c in copies(b, slot):
        c.wait()

    i128 = lax.broadcasted_iota(jnp.int32, (128,), 0)
    i384 = lax.broadcasted_iota(jnp.int32, (384,), 0)
    for k in range(_BEAM):
        p_k = 32 * lax.rem(2 * b + k, 4)
        hoff = pl.multiple_of(soff + k * _SLOT, 1024)
        xb[pl.ds(hoff, 128)] = jnp.where(i128 < p_k, -jnp.inf, xb[pl.ds(hoff, 128)])
        toff = pl.multiple_of(soff + k * _SLOT + _SLOT - 384, 128)
        xb[pl.ds(toff, 384)] = jnp.where(
            i384 >= p_k + 32, -jnp.inf, xb[pl.ds(toff, 384)]
        )

    x = xb[pl.ds(soff, _BEAM * _SLOT)].reshape(_BEAM, _RS, 128)
    bls = bls_ref[0]
    seqt = seqt_ref[0]

    big = jnp.int32(2**31 - 1)
    neg = jnp.float32(-jnp.inf)

    beam3 = lax.broadcasted_iota(jnp.int32, (_BEAM, 1, 1), 0)
    p_arr = 32 * lax.rem(2 * b + beam3, 4)
    vflat = (
        lax.broadcasted_iota(jnp.int32, (_BEAM, _RS, 128), 1) * 128
        + lax.broadcasted_iota(jnp.int32, (_BEAM, _RS, 128), 2)
        - p_arr
    )

    p0 = jnp.max(x, axis=1)
    m = jnp.max(p0, axis=1, keepdims=True)
    e = jnp.exp(x - m[:, :, None])
    s = jnp.sum(jnp.sum(e, axis=1), axis=1, keepdims=True)
    shift = bls - jnp.log(s)

    cp = (p0 - m) + shift

    cf = jnp.min(jnp.where(x == p0[:, None, :], vflat, big), axis=1)
    cff = lax.broadcasted_iota(jnp.int32, (_BEAM, 128), 0) * _VOCAB + cf

    lane16 = lax.broadcasted_iota(jnp.int32, (1, _LANES), 1)
    beam_pc = lax.broadcasted_iota(jnp.int32, (_BEAM, 128), 0)
    lane_pc = lax.broadcasted_iota(jnp.int32, (_BEAM, 128), 1)
    beam10 = lax.broadcasted_iota(jnp.int32, (_BEAM, 1), 0)
    lane_slab = lax.broadcasted_iota(jnp.int32, (_RS, 128), 1)
    rl_slab = (
        lax.broadcasted_iota(jnp.int32, (_RS, 128), 0) * 128
        + lax.broadcasted_iota(jnp.int32, (_RS, 128), 1)
    )

    vals = jnp.zeros((1, _LANES), jnp.float32)
    fidx = jnp.zeros((1, _LANES), jnp.int32)
    for i in range(_BEAM):
        gm = jnp.max(cp)
        fi = jnp.min(jnp.where(cp == gm, cff, big))
        vals = jnp.where(lane16 == i, gm, vals)
        fidx = jnp.where(lane16 == i, fi, fidx)
        if i + 1 == _BEAM:
            break

        kstar = fi // _VOCAB
        vstar = lax.rem(fi, _VOCAB)
        p_k = 32 * lax.rem(2 * b + kstar, 4)
        lstar = lax.rem(vstar + p_k, 128)
        ksel = beam10 == kstar
        m_k = jnp.sum(jnp.where(ksel, m, 0.0))
        sh_k = jnp.sum(jnp.where(ksel, shift, 0.0))

        off = pl.multiple_of(soff + kstar * _SLOT, 1024)
        slab = xb[pl.ds(off, _SLOT)].reshape(_RS, 128)
        vf_s = rl_slab - p_k
        cs = (slab - m_k) + sh_k
        ff = kstar * _VOCAB + vf_s
        elig = (lane_slab == lstar) & ((cs < gm) | ((cs == gm) & (ff > fi)))
        newv = jnp.max(jnp.where(elig, cs, neg))
        newf = jnp.min(jnp.where(elig & (cs == newv), ff, big))

        colmask = (beam_pc == kstar) & (lane_pc == lstar)
        cp = jnp.where(colmask, newv, cp)
        cff = jnp.where(colmask, newf, cff)

    beam_ix = fidx // _VOCAB
    sel_ix = fidx % _VOCAB

    new_prev_i = jnp.zeros((_TPREV, _LANES), jnp.int32)
    for k in range(_BEAM):
        col = seqt[:, k : k + 1]
        new_prev_i = jnp.where(beam_ix == k, col, new_prev_i)

    seq_out_ref[0] = jnp.concatenate([new_prev_i, sel_ix], axis=0)
    bls_out_ref[0] = vals


def kernel(logprobs, beam_logprobs_sum, beam_seq):
    bls = beam_logprobs_sum.reshape(_BATCH, _BEAM, 1)
    seqt = beam_seq.transpose(0, 2, 1)

    seq_out, bls_out = pl.pallas_call(
        _beam_step_body,
        grid=(_BATCH,),
        in_specs=[
            pl.BlockSpec(memory_space=pl.ANY),
            pl.BlockSpec((1, _BEAM, 1), lambda b: (b, 0, 0)),
            pl.BlockSpec((1, _TPREV, _BEAM), lambda b: (b, 0, 0)),
        ],
        out_specs=[
            pl.BlockSpec((1, _TPREV + 1, _LANES), lambda b: (b, 0, 0)),
            pl.BlockSpec((1, 1, _LANES), lambda b: (b, 0, 0)),
        ],
        out_shape=[
            jax.ShapeDtypeStruct((_BATCH, _TPREV + 1, _LANES), jnp.int32),
            jax.ShapeDtypeStruct((_BATCH, 1, _LANES), jnp.float32),
        ],
        scratch_shapes=[
            pltpu.VMEM((_NBUF * _BEAM * _SLOT,), jnp.float32),
            pltpu.SemaphoreType.DMA((_NBUF,)),
        ],
    )(logprobs.reshape(-1), bls, seqt)

    new_seq = seq_out.transpose(0, 2, 1)[:, :_BEAM, :]
    new_bls = bls_out[:, 0, :_BEAM]
    return new_seq, new_bls
